# Initial kernel scaffold; baseline (speedup 1.0000x reference)
import jax
import jax.numpy as jnp
from jax.experimental import pallas as pl


def kernel(x_list, edge_index, W0, b0, W1, b1, Wf, bf):
    raise NotImplementedError("write your pallas kernel here")


# speedup vs baseline: 4.0484x; 4.0484x over previous
import functools

import jax
import jax.numpy as jnp
from jax import lax
from jax.experimental import pallas as pl
from jax.experimental.pallas import tpu as pltpu
from jax.experimental.pallas import tpu_sc as plsc

_N = 10000
_NP = 10240
_E = 320000
_BLK = 128
_NW = 32
_BPW = 80
_EP = _NW * _BPW * _BLK
_NS = 16
_RPS = _NP // _NS
_ALPHA = 0.1
_K = 10
_C = 40


def _make_sc_scatter(d):
    mesh = plsc.VectorSubcoreMesh(core_axis_name="c", subcore_axis_name="s")

    @functools.partial(
        pl.kernel,
        mesh=mesh,
        out_type=jax.ShapeDtypeStruct((2 * _NP, d), jnp.float32),
        scratch_types=[
            pltpu.VMEM((_BPW, _BLK), jnp.int32),
            pltpu.VMEM((_BPW, _BLK), jnp.int32),
            pltpu.VMEM((_BLK, d), jnp.float32),
            pltpu.VMEM_SHARED((_NP, d), jnp.float32),
            pltpu.SemaphoreType.DMA,
        ],
    )
    def sc_scatter(zt_hbm, src_hbm, dst_hbm, zeros_hbm, out_hbm,
                   src_v, dst_v, rows_v, acc_sh, sem):
        c = lax.axis_index("c")
        s = lax.axis_index("s")
        w = s * 2 + c
        pltpu.sync_copy(zeros_hbm.at[pl.ds(s * _RPS, _RPS)],
                        acc_sh.at[pl.ds(s * _RPS, _RPS)])
        pltpu.sync_copy(src_hbm.at[w], src_v)
        pltpu.sync_copy(dst_hbm.at[w], dst_v)
        plsc.subcore_barrier()

        def body(j, carry):
            pltpu.async_copy(zt_hbm.at[src_v.at[j]], rows_v, sem).wait()
            pltpu.sync_copy(rows_v, acc_sh.at[dst_v.at[j]], add=True)
            return carry

        lax.fori_loop(0, _BPW, body, 0)
        plsc.subcore_barrier()
        pltpu.sync_copy(acc_sh.at[pl.ds(s * _RPS, _RPS)],
                        out_hbm.at[pl.ds(c * _NP + s * _RPS, _RPS)])

    return sc_scatter


_sc_scatter_128 = _make_sc_scatter(128)


def _sc_partials(zt, src3, dst3, zeros, d):
    p = _sc_scatter_128(zt, src3, dst3, zeros)
    return p[:_NP], p[_NP:]


_ROWBLK = 2048


def _tc_linear(x, w, b):
    din, dout = w.shape

    def body(x_ref, w_ref, b_ref, o_ref):
        o_ref[...] = jnp.dot(x_ref[...], w_ref[...],
                             preferred_element_type=jnp.float32) + b_ref[0, :][None, :]

    return pl.pallas_call(
        body,
        grid=(_NP // _ROWBLK,),
        in_specs=[pl.BlockSpec((_ROWBLK, din), lambda i: (i, 0)),
                  pl.BlockSpec((din, dout), lambda i: (0, 0)),
                  pl.BlockSpec((8, dout), lambda i: (0, 0))],
        out_specs=pl.BlockSpec((_ROWBLK, dout), lambda i: (i, 0)),
        out_shape=jax.ShapeDtypeStruct((_NP, dout), jnp.float32),
    )(x, w, jnp.broadcast_to(b[None, :], (8, dout)))


def _tc_dinv(d0, d1):
    def body(a_ref, b_ref, o_ref):
        o_ref[...] = lax.rsqrt(a_ref[...] + b_ref[...] + 1.0)

    return pl.pallas_call(
        body,
        grid=(_NP // _ROWBLK,),
        in_specs=[pl.BlockSpec((_ROWBLK, 128), lambda i: (i, 0))] * 2,
        out_specs=pl.BlockSpec((_ROWBLK, 128), lambda i: (i, 0)),
        out_shape=jax.ShapeDtypeStruct((_NP, 128), jnp.float32),
    )(d0, d1)


def _tc_mul(a, b):
    d = a.shape[1]

    def body(a_ref, b_ref, o_ref):
        o_ref[...] = a_ref[...] * b_ref[...]

    return pl.pallas_call(
        body,
        grid=(_NP // _ROWBLK,),
        in_specs=[pl.BlockSpec((_ROWBLK, d), lambda i: (i, 0))] * 2,
        out_specs=pl.BlockSpec((_ROWBLK, d), lambda i: (i, 0)),
        out_shape=jax.ShapeDtypeStruct((_NP, d), jnp.float32),
    )(a, b)


def _tc_combine(p0, p1, z, h, dv, relu):
    d = z.shape[1]

    def body(p0_ref, p1_ref, z_ref, h_ref, dv_ref, zo_ref, zto_ref):
        dvv = dv_ref[...]
        az = dvv * (p0_ref[...] + p1_ref[...]) + dvv * dvv * z_ref[...]
        zn = (1.0 - _ALPHA) * az + _ALPHA * h_ref[...]
        if relu:
            zn = jnp.maximum(zn, 0.0)
        zo_ref[...] = zn
        zto_ref[...] = dvv * zn

    return pl.pallas_call(
        body,
        grid=(_NP // _ROWBLK,),
        in_specs=[pl.BlockSpec((_ROWBLK, d), lambda i: (i, 0))] * 5,
        out_specs=[pl.BlockSpec((_ROWBLK, d), lambda i: (i, 0))] * 2,
        out_shape=[jax.ShapeDtypeStruct((_NP, d), jnp.float32)] * 2,
    )(p0, p1, z, h, dv)


def _propagate(h, src3, dst3, zeros, dv, d, relu_last):
    z = h
    zt = _tc_mul(dv, h)
    for step in range(_K):
        p0, p1 = _sc_partials(zt, src3, dst3, zeros, d)
        z, zt = _tc_combine(p0, p1, z, h, dv, relu=(relu_last and step == _K - 1))
    return z


def kernel(x_list, edge_index, W0, b0, W1, b1, Wf, bf):
    src = edge_index[0].astype(jnp.int32)
    dst = edge_index[1].astype(jnp.int32)
    pad = jnp.full((_EP - _E,), _NP - 1, jnp.int32)
    src3 = jnp.concatenate([src, pad]).reshape(_NW, _BPW, _BLK)
    dst3 = jnp.concatenate([dst, pad]).reshape(_NW, _BPW, _BLK)

    zeros128 = jnp.zeros((_NP, 128), jnp.float32)

    ones_tab = jnp.concatenate(
        [jnp.ones((_N, 128), jnp.float32), jnp.zeros((_NP - _N, 128), jnp.float32)])
    d0, d1 = _sc_partials(ones_tab, src3, dst3, zeros128, 128)
    dv128 = _tc_dinv(d0, d1)

    hidden = []
    Ws = [(W0, b0), (W1, b1)]
    for k in range(2):
        xp = jnp.pad(x_list[k], ((0, _NP - _N), (0, 0)))
        h = _tc_linear(xp, Ws[k][0], Ws[k][1])
        hidden.append(_propagate(h, src3, dst3, zeros128, dv128, 128, relu_last=True))

    xcat = jnp.concatenate(hidden, axis=1)
    Wfp = jnp.pad(Wf, ((0, 0), (0, 128 - _C)))
    bfp = jnp.pad(bf, ((0, 128 - _C),))
    h3 = _tc_linear(xcat, Wfp, bfp)
    z3 = _propagate(h3, src3, dst3, zeros128, dv128, 128, relu_last=False)
    return z3[:_N, :_C]


# speedup vs baseline: 4.2923x; 1.0603x over previous
import functools

import jax
import jax.numpy as jnp
from jax import lax
from jax.experimental import pallas as pl
from jax.experimental.pallas import tpu as pltpu
from jax.experimental.pallas import tpu_sc as plsc

_N = 10000
_NP = 10240
_E = 320000
_BLK = 128
_NW = 32
_BPW = 80
_EP = _NW * _BPW * _BLK
_NS = 16
_NBUF = 2
_CHUNK = 16
_RPS = _NP // _NS
_ALPHA = 0.1
_K = 10
_C = 40


def _make_sc_scatter(d):
    mesh = plsc.VectorSubcoreMesh(core_axis_name="c", subcore_axis_name="s")

    @functools.partial(
        pl.kernel,
        mesh=mesh,
        out_type=jax.ShapeDtypeStruct((2 * _NP, d), jnp.float32),
        scratch_types=[
            pltpu.VMEM((_CHUNK, _BLK), jnp.int32),
            pltpu.VMEM((_CHUNK, _BLK), jnp.int32),
            *[pltpu.VMEM((_BLK, d), jnp.float32) for _ in range(_NBUF)],
            pltpu.VMEM_SHARED((_NP, d), jnp.float32),
            pltpu.SemaphoreType.DMA,
        ],
    )
    def sc_scatter(zt_hbm, src_hbm, dst_hbm, zeros_hbm, out_hbm,
                   src_v, dst_v, r0, r1, acc_sh, gsem):
        rows = [r0, r1]
        c = lax.axis_index("c")
        s = lax.axis_index("s")
        w = s * 2 + c
        pltpu.sync_copy(zeros_hbm.at[pl.ds(s * _RPS, _RPS)],
                        acc_sh.at[pl.ds(s * _RPS, _RPS)])
        plsc.subcore_barrier()

        def chunk_body(cc, carry):
            pltpu.sync_copy(src_hbm.at[w].at[pl.ds(cc * _CHUNK, _CHUNK)], src_v)
            pltpu.sync_copy(dst_hbm.at[w].at[pl.ds(cc * _CHUNK, _CHUNK)], dst_v)

            def body(t, c2):
                gets = [pltpu.async_copy(zt_hbm.at[src_v.at[_NBUF * t + b]],
                                         rows[b], gsem)
                        for b in range(_NBUF)]
                for b in range(_NBUF):
                    gets[b].wait()
                    pltpu.sync_copy(rows[b],
                                    acc_sh.at[dst_v.at[_NBUF * t + b]],
                                    add=True)
                return c2

            lax.fori_loop(0, _CHUNK // _NBUF, body, carry)
            return carry

        lax.fori_loop(0, _BPW // _CHUNK, chunk_body, 0)
        plsc.subcore_barrier()
        pltpu.sync_copy(acc_sh.at[pl.ds(s * _RPS, _RPS)],
                        out_hbm.at[pl.ds(c * _NP + s * _RPS, _RPS)])

    return sc_scatter


_sc_scatter_128 = _make_sc_scatter(128)


def _sc_partials(zt, src3, dst3, zeros, d):
    p = _sc_scatter_128(zt, src3, dst3, zeros)
    return p[:_NP], p[_NP:]


_ROWBLK = 2048


def _tc_linear(x, w, b):
    din, dout = w.shape

    def body(x_ref, w_ref, b_ref, o_ref):
        o_ref[...] = jnp.dot(x_ref[...], w_ref[...],
                             preferred_element_type=jnp.float32) + b_ref[0, :][None, :]

    return pl.pallas_call(
        body,
        grid=(_NP // _ROWBLK,),
        in_specs=[pl.BlockSpec((_ROWBLK, din), lambda i: (i, 0)),
                  pl.BlockSpec((din, dout), lambda i: (0, 0)),
                  pl.BlockSpec((8, dout), lambda i: (0, 0))],
        out_specs=pl.BlockSpec((_ROWBLK, dout), lambda i: (i, 0)),
        out_shape=jax.ShapeDtypeStruct((_NP, dout), jnp.float32),
    )(x, w, jnp.broadcast_to(b[None, :], (8, dout)))


def _tc_dinv(d0, d1):
    def body(a_ref, b_ref, o_ref):
        o_ref[...] = lax.rsqrt(a_ref[...] + b_ref[...] + 1.0)

    return pl.pallas_call(
        body,
        grid=(_NP // _ROWBLK,),
        in_specs=[pl.BlockSpec((_ROWBLK, 128), lambda i: (i, 0))] * 2,
        out_specs=pl.BlockSpec((_ROWBLK, 128), lambda i: (i, 0)),
        out_shape=jax.ShapeDtypeStruct((_NP, 128), jnp.float32),
    )(d0, d1)


def _tc_mul(a, b):
    d = a.shape[1]

    def body(a_ref, b_ref, o_ref):
        o_ref[...] = a_ref[...] * b_ref[...]

    return pl.pallas_call(
        body,
        grid=(_NP // _ROWBLK,),
        in_specs=[pl.BlockSpec((_ROWBLK, d), lambda i: (i, 0))] * 2,
        out_specs=pl.BlockSpec((_ROWBLK, d), lambda i: (i, 0)),
        out_shape=jax.ShapeDtypeStruct((_NP, d), jnp.float32),
    )(a, b)


def _tc_combine(p0, p1, z, h, dv, relu):
    d = z.shape[1]

    def body(p0_ref, p1_ref, z_ref, h_ref, dv_ref, zo_ref, zto_ref):
        dvv = dv_ref[...]
        az = dvv * (p0_ref[...] + p1_ref[...]) + dvv * dvv * z_ref[...]
        zn = (1.0 - _ALPHA) * az + _ALPHA * h_ref[...]
        if relu:
            zn = jnp.maximum(zn, 0.0)
        zo_ref[...] = zn
        zto_ref[...] = dvv * zn

    return pl.pallas_call(
        body,
        grid=(_NP // _ROWBLK,),
        in_specs=[pl.BlockSpec((_ROWBLK, d), lambda i: (i, 0))] * 5,
        out_specs=[pl.BlockSpec((_ROWBLK, d), lambda i: (i, 0))] * 2,
        out_shape=[jax.ShapeDtypeStruct((_NP, d), jnp.float32)] * 2,
    )(p0, p1, z, h, dv)


def _propagate(h, src3, dst3, zeros, dv, d, relu_last):
    z = h
    zt = _tc_mul(dv, h)
    for step in range(_K):
        p0, p1 = _sc_partials(zt, src3, dst3, zeros, d)
        z, zt = _tc_combine(p0, p1, z, h, dv, relu=(relu_last and step == _K - 1))
    return z


def kernel(x_list, edge_index, W0, b0, W1, b1, Wf, bf):
    src = edge_index[0].astype(jnp.int32)
    dst = edge_index[1].astype(jnp.int32)
    pad = jnp.full((_EP - _E,), _NP - 1, jnp.int32)
    src3 = jnp.concatenate([src, pad]).reshape(_NW, _BPW, _BLK)
    dst3 = jnp.concatenate([dst, pad]).reshape(_NW, _BPW, _BLK)

    zeros128 = jnp.zeros((_NP, 128), jnp.float32)

    ones_tab = jnp.concatenate(
        [jnp.ones((_N, 128), jnp.float32), jnp.zeros((_NP - _N, 128), jnp.float32)])
    d0, d1 = _sc_partials(ones_tab, src3, dst3, zeros128, 128)
    dv128 = _tc_dinv(d0, d1)

    hidden = []
    Ws = [(W0, b0), (W1, b1)]
    for k in range(2):
        xp = jnp.pad(x_list[k], ((0, _NP - _N), (0, 0)))
        h = _tc_linear(xp, Ws[k][0], Ws[k][1])
        hidden.append(_propagate(h, src3, dst3, zeros128, dv128, 128, relu_last=True))

    xcat = jnp.concatenate(hidden, axis=1)
    Wfp = jnp.pad(Wf, ((0, 0), (0, 128 - _C)))
    bfp = jnp.pad(bf, ((0, 128 - _C),))
    h3 = _tc_linear(xcat, Wfp, bfp)
    z3 = _propagate(h3, src3, dst3, zeros128, dv128, 128, relu_last=False)
    return z3[:_N, :_C]
